# speedup vs baseline: 2.1744x; 2.1744x over previous
import functools

import jax
import jax.numpy as jnp
from jax import lax
from jax.experimental import pallas as pl
from jax.experimental.pallas import tpu as pltpu
from jax.experimental.pallas import tpu_sc as plsc

D = 1024
B = 4 * 8192
NC = 2
NS = 16
NW = NC * NS
BPW = B // NW
C = 64
NCH = BPW // C


def _body(table_hbm, idx_hbm, out_hbm, idx_v, buf, sem):
    wid = lax.axis_index("s") * NC + lax.axis_index("c")
    base = wid * BPW
    pltpu.sync_copy(idx_hbm.at[pl.ds(base, BPW)], idx_v)

    def chunk(g, carry):
        off = g * C
        pltpu.async_copy(
            table_hbm.at[idx_v.at[pl.ds(off, C)]], buf, sem
        ).wait()
        pltpu.sync_copy(buf, out_hbm.at[pl.ds(base + off, C)])
        return carry

    lax.fori_loop(0, NCH, chunk, 0)


_gather = functools.partial(
    pl.kernel,
    out_type=jax.ShapeDtypeStruct((B, D), jnp.float32),
    mesh=plsc.VectorSubcoreMesh(core_axis_name="c", subcore_axis_name="s"),
    scratch_types=[
        pltpu.VMEM((BPW,), jnp.int32),
        pltpu.VMEM((C, D), jnp.float32),
        pltpu.SemaphoreType.DMA,
    ],
)(_body)


@jax.jit
def kernel(src_seq, pos_table):
    idx = src_seq.reshape(-1).astype(jnp.int32)
    out = _gather(pos_table, idx)
    return out.reshape(src_seq.shape + (D,))


# speedup vs baseline: 2.2478x; 1.0337x over previous
import functools

import jax
import jax.numpy as jnp
from jax import lax
from jax.experimental import pallas as pl
from jax.experimental.pallas import tpu as pltpu
from jax.experimental.pallas import tpu_sc as plsc

D = 1024
B = 4 * 8192
NC = 2
NS = 16
NW = NC * NS
BPW = B // NW
C = 32
NCH = BPW // C


def _body(table_hbm, idx_hbm, out_hbm, idx_v, buf0, buf1, gs0, gs1, ws0, ws1):
    wid = lax.axis_index("s") * NC + lax.axis_index("c")
    base = wid * BPW
    pltpu.sync_copy(idx_hbm.at[pl.ds(base, BPW)], idx_v)

    bufs = (buf0, buf1)
    gs = (gs0, gs1)
    ws = (ws0, ws1)

    def gather(g, b):
        return pltpu.async_copy(
            table_hbm.at[idx_v.at[pl.ds(g * C, C)]], bufs[b], gs[b]
        )

    def write(g, b):
        return pltpu.async_copy(
            bufs[b], out_hbm.at[pl.ds(base + g * C, C)], ws[b]
        )

    pg = [None, None]
    pw = [None, None]
    pg[0] = gather(0, 0)
    for g in range(NCH):
        b = g & 1
        o = b ^ 1
        pg[b].wait()
        pg[b] = None
        pw[b] = write(g, b)
        if g + 1 < NCH:
            if pw[o] is not None:
                pw[o].wait()
            pg[o] = gather(g + 1, o)
    for b in (0, 1):
        if pw[b] is not None:
            pw[b].wait()


_gather = functools.partial(
    pl.kernel,
    out_type=jax.ShapeDtypeStruct((B, D), jnp.float32),
    mesh=plsc.VectorSubcoreMesh(core_axis_name="c", subcore_axis_name="s"),
    scratch_types=[
        pltpu.VMEM((BPW,), jnp.int32),
        pltpu.VMEM((C, D), jnp.float32),
        pltpu.VMEM((C, D), jnp.float32),
        pltpu.SemaphoreType.DMA,
        pltpu.SemaphoreType.DMA,
        pltpu.SemaphoreType.DMA,
        pltpu.SemaphoreType.DMA,
    ],
)(_body)


@jax.jit
def kernel(src_seq, pos_table):
    idx = src_seq.reshape(-1).astype(jnp.int32)
    out = _gather(pos_table, idx)
    return out.reshape(src_seq.shape + (D,))


# speedup vs baseline: 2.3324x; 1.0376x over previous
import functools

import jax
import jax.numpy as jnp
from jax import lax
from jax.experimental import pallas as pl
from jax.experimental.pallas import tpu as pltpu
from jax.experimental.pallas import tpu_sc as plsc

D = 1024
B = 4 * 8192
NC = 2
NS = 16
NW = NC * NS
BPW = B // NW
C = 32
NCH = BPW // C
NBUF = 3


def _body(table_hbm, idx_hbm, out_hbm, idx_v, *rest):
    bufs = rest[:NBUF]
    gs = rest[NBUF:2 * NBUF]
    ws = rest[2 * NBUF:3 * NBUF]

    wid = lax.axis_index("s") * NC + lax.axis_index("c")
    base = wid * BPW
    pltpu.sync_copy(idx_hbm.at[pl.ds(base, BPW)], idx_v)

    def gather(g, b):
        return pltpu.async_copy(
            table_hbm.at[idx_v.at[pl.ds(g * C, C)]], bufs[b], gs[b]
        )

    def write(g, b):
        return pltpu.async_copy(
            bufs[b], out_hbm.at[pl.ds(base + g * C, C)], ws[b]
        )

    pg = [None] * NBUF
    pw = [None] * NBUF
    for v in range(NCH + NBUF - 1):
        if v < NCH:
            b = v % NBUF
            if pw[b] is not None:
                pw[b].wait()
                pw[b] = None
            pg[b] = gather(v, b)
        gc = v - (NBUF - 1)
        if gc >= 0:
            b = gc % NBUF
            pg[b].wait()
            pg[b] = None
            pw[b] = write(gc, b)
    for b in range(NBUF):
        if pw[b] is not None:
            pw[b].wait()


_gather = functools.partial(
    pl.kernel,
    out_type=jax.ShapeDtypeStruct((B, D), jnp.float32),
    mesh=plsc.VectorSubcoreMesh(core_axis_name="c", subcore_axis_name="s"),
    scratch_types=(
        [pltpu.VMEM((BPW,), jnp.int32)]
        + [pltpu.VMEM((C, D), jnp.float32) for _ in range(NBUF)]
        + [pltpu.SemaphoreType.DMA for _ in range(2 * NBUF)]
    ),
)(_body)


@jax.jit
def kernel(src_seq, pos_table):
    idx = src_seq.reshape(-1).astype(jnp.int32)
    out = _gather(pos_table, idx)
    return out.reshape(src_seq.shape + (D,))
